# baseline (speedup 1.0000x reference)
import functools

import jax
import jax.numpy as jnp
from jax import lax
from jax.experimental import pallas as pl
from jax.experimental.pallas import tpu as pltpu
from jax.experimental.pallas import tpu_sc as plsc

EMBED = 16
IDX_CHUNK = 128
LANE_BLK = 8192
REG_COEF = 0.001


def _tc_sweep_body(wv, src, tdst, sdst):
    one = jnp.ones((1, EMBED), jnp.float32)
    blk = src[...]
    w = wv[0:1, :]
    tdst[...] = jnp.dot(w, blk, preferred_element_type=jnp.float32)[0]
    sdst[...] = jnp.dot(one, blk * blk, preferred_element_type=jnp.float32)[0]


@functools.lru_cache(maxsize=None)
def _build_tc_sweep(n: int):
    grid = (pl.cdiv(n, LANE_BLK),)
    tbl_spec = pl.BlockSpec((EMBED, LANE_BLK), lambda c: (0, c),
                            pipeline_mode=pl.Buffered(buffer_count=2))
    vec_spec = pl.BlockSpec((LANE_BLK,), lambda c: (c,),
                            pipeline_mode=pl.Buffered(buffer_count=2))
    return pl.pallas_call(
        _tc_sweep_body,
        grid=grid,
        in_specs=[pl.BlockSpec((8, EMBED), lambda c: (0, 0)), tbl_spec],
        out_specs=[vec_spec, vec_spec],
        out_shape=[jax.ShapeDtypeStruct((n,), jnp.float32)] * 2,
        compiler_params=pltpu.CompilerParams(
            dimension_semantics=("arbitrary",)),
    )


@functools.lru_cache(maxsize=None)
def _build_sc_lookup(batch: int):
    info = plsc.get_sparse_core_info()
    ncores, nsub, lanes = info.num_cores, info.num_subcores, info.num_lanes
    nw = ncores * nsub
    bpw = batch // nw
    nch = bpw // IDX_CHUNK
    nblk = bpw // lanes

    mesh = plsc.VectorSubcoreMesh(core_axis_name="c", subcore_axis_name="s")

    @functools.partial(
        pl.kernel,
        mesh=mesh,
        out_type=[
            jax.ShapeDtypeStruct((batch,), jnp.float32),
            jax.ShapeDtypeStruct((nw, EMBED), jnp.float32),
        ],
        scratch_types=[
            pltpu.VMEM((nch, IDX_CHUNK), jnp.int32),
            pltpu.VMEM((bpw,), jnp.float32),
            pltpu.VMEM((bpw,), jnp.float32),
            pltpu.VMEM((EMBED,), jnp.float32),
            pltpu.SemaphoreType.DMA,
        ],
        compiler_params=pltpu.CompilerParams(
            needs_layout_passes=False, use_tc_tiling_on_sc=False),
    )
    def sc_lookup(xs2, tvec, svec, out_y, out_ss,
                  idx, g_t, g_s, ss_v, sem):
        wid = lax.axis_index("s") * ncores + lax.axis_index("c")
        base = wid * bpw

        pltpu.sync_copy(xs2.at[pl.ds(wid * nch, nch)], idx)

        copies = []
        for vec, dst in ((tvec, g_t), (svec, g_s)):
            for j in range(nch):
                copies.append(pltpu.async_copy(
                    vec.at[idx.at[j]],
                    dst.at[pl.ds(j * IDX_CHUNK, IDX_CHUNK)],
                    sem))
        for c in copies:
            c.wait()

        iota = lax.iota(jnp.int32, lanes)
        zero = jnp.zeros((lanes,), jnp.float32)

        def body(b, acc):
            return acc + g_s[pl.ds(b * lanes, lanes)]

        acc = lax.fori_loop(0, nblk, body, zero)

        ssvec = jnp.where(iota == 0, jnp.sum(acc), 0.0)
        ss_v[...] = ssvec.astype(jnp.float32)

        pltpu.sync_copy(g_t, out_y.at[pl.ds(base, bpw)])
        pltpu.sync_copy(ss_v, out_ss.at[wid])

    return sc_lookup


def kernel(ps, qs, rs, P, Q, R, wp, bp, wq, bq, wr, br, W):
    batch = ps.shape[0]
    wc = W[0].astype(jnp.float32)
    c = jnp.sqrt(jnp.sum(wc * wc))
    wc = jnp.where(c > 1.0, wc / c, wc)

    sweep = _build_tc_sweep(P.shape[0])
    lookup = _build_sc_lookup(batch)

    pad = jnp.zeros((7, EMBED), jnp.float32)

    def one_table(idxs, tbl, w_scalar):
        wv = jnp.concatenate([(w_scalar * wc)[None, :], pad])
        t, s = sweep(wv, tbl.T)
        xs2 = idxs.astype(jnp.int32).reshape(-1, IDX_CHUNK)
        y, ss = lookup(xs2, t, s)
        return y, jnp.sum(ss[:, 0])

    y_p, ss_p = one_table(ps, P, wp[0, 0])
    y_q, ss_q = one_table(qs, Q, wq[0, 0])
    y_r, ss_r = one_table(rs, R, wr[0, 0])

    bias = (bp[0] + bq[0] + br[0]) * jnp.sum(wc)
    inferences = (y_p + y_q + y_r + bias).reshape(batch, 1)
    regs = REG_COEF * (jnp.sqrt(ss_p) + jnp.sqrt(ss_q) + jnp.sqrt(ss_r))
    return (inferences, regs)


# speedup vs baseline: 2.3209x; 2.3209x over previous
import functools

import jax
import jax.numpy as jnp
from jax import lax
from jax.experimental import pallas as pl
from jax.experimental.pallas import tpu as pltpu
from jax.experimental.pallas import tpu_sc as plsc

EMBED = 16
IDX_CHUNK = 128
LANE_BLK = 16384
REG_COEF = 0.001


def _tc_sweep_body(wv, pt, qt, rt, tp, sp, tq, sq, tr, sr):
    one = jnp.ones((1, EMBED), jnp.float32)
    for t, (src, tdst, sdst) in enumerate(
        ((pt, tp, sp), (qt, tq, sq), (rt, tr, sr))):
        blk = src[...]
        w = wv[t:t + 1, :]
        tdst[...] = jnp.dot(w, blk, preferred_element_type=jnp.float32)[0]
        sdst[...] = jnp.dot(one, blk * blk,
                            preferred_element_type=jnp.float32)[0]


@functools.lru_cache(maxsize=None)
def _build_tc_sweep(n: int):
    grid = (pl.cdiv(n, LANE_BLK),)
    tbl_spec = pl.BlockSpec((EMBED, LANE_BLK), lambda c: (0, c),
                            pipeline_mode=pl.Buffered(buffer_count=2))
    vec_spec = pl.BlockSpec((LANE_BLK,), lambda c: (c,),
                            pipeline_mode=pl.Buffered(buffer_count=2))
    return pl.pallas_call(
        _tc_sweep_body,
        grid=grid,
        in_specs=[pl.BlockSpec((4, EMBED), lambda c: (0, 0)),
                  tbl_spec, tbl_spec, tbl_spec],
        out_specs=[vec_spec] * 6,
        out_shape=[jax.ShapeDtypeStruct((n,), jnp.float32)] * 6,
        compiler_params=pltpu.CompilerParams(
            dimension_semantics=("arbitrary",)),
    )


@functools.lru_cache(maxsize=None)
def _build_sc_lookup(batch: int):
    info = plsc.get_sparse_core_info()
    ncores, nsub, lanes = info.num_cores, info.num_subcores, info.num_lanes
    nw = ncores * nsub
    bpw = batch // nw
    nch = bpw // IDX_CHUNK
    nblk = bpw // lanes

    mesh = plsc.VectorSubcoreMesh(core_axis_name="c", subcore_axis_name="s")

    @functools.partial(
        pl.kernel,
        mesh=mesh,
        out_type=[
            jax.ShapeDtypeStruct((batch,), jnp.float32),
            jax.ShapeDtypeStruct((nw, EMBED), jnp.float32),
        ],
        scratch_types=[
            pltpu.VMEM((nch, IDX_CHUNK), jnp.int32),
            pltpu.VMEM((nch, IDX_CHUNK), jnp.int32),
            pltpu.VMEM((nch, IDX_CHUNK), jnp.int32),
            pltpu.VMEM((bpw,), jnp.float32),
            pltpu.VMEM((bpw,), jnp.float32),
            pltpu.VMEM((bpw,), jnp.float32),
            pltpu.VMEM((bpw,), jnp.float32),
            pltpu.VMEM((bpw,), jnp.float32),
            pltpu.VMEM((bpw,), jnp.float32),
            pltpu.VMEM((bpw,), jnp.float32),
            pltpu.VMEM((EMBED,), jnp.float32),
            pltpu.SemaphoreType.DMA,
        ],
        compiler_params=pltpu.CompilerParams(
            needs_layout_passes=False, use_tc_tiling_on_sc=False),
    )
    def sc_lookup(ps2, qs2, rs2, tp, sp, tq, sq, tr, sr,
                  out_y, out_ss,
                  idx_p, idx_q, idx_r,
                  g_tp, g_sp, g_tq, g_sq, g_tr, g_sr,
                  y_v, ss_v, sem):
        wid = lax.axis_index("s") * ncores + lax.axis_index("c")
        base = wid * bpw

        pltpu.sync_copy(ps2.at[pl.ds(wid * nch, nch)], idx_p)
        pltpu.sync_copy(qs2.at[pl.ds(wid * nch, nch)], idx_q)
        pltpu.sync_copy(rs2.at[pl.ds(wid * nch, nch)], idx_r)

        copies = []
        for vec, idx, dst in ((tp, idx_p, g_tp), (sp, idx_p, g_sp),
                              (tq, idx_q, g_tq), (sq, idx_q, g_sq),
                              (tr, idx_r, g_tr), (sr, idx_r, g_sr)):
            for j in range(nch):
                copies.append(pltpu.async_copy(
                    vec.at[idx.at[j]],
                    dst.at[pl.ds(j * IDX_CHUNK, IDX_CHUNK)],
                    sem))
        for c in copies:
            c.wait()

        iota = lax.iota(jnp.int32, lanes)
        zero = jnp.zeros((lanes,), jnp.float32)

        def body(b, carry):
            ap, aq, ar = carry
            sl = pl.ds(b * lanes, lanes)
            y_v[sl] = g_tp[sl] + g_tq[sl] + g_tr[sl]
            return ap + g_sp[sl], aq + g_sq[sl], ar + g_sr[sl]

        ap, aq, ar = lax.fori_loop(0, nblk, body, (zero, zero, zero))

        ssvec = (jnp.where(iota == 0, jnp.sum(ap), 0.0)
                 + jnp.where(iota == 1, jnp.sum(aq), 0.0)
                 + jnp.where(iota == 2, jnp.sum(ar), 0.0))
        ss_v[...] = ssvec.astype(jnp.float32)

        pltpu.sync_copy(y_v, out_y.at[pl.ds(base, bpw)])
        pltpu.sync_copy(ss_v, out_ss.at[wid])

    return sc_lookup


def kernel(ps, qs, rs, P, Q, R, wp, bp, wq, bq, wr, br, W):
    batch = ps.shape[0]
    n = P.shape[0]
    wc = W[0].astype(jnp.float32)
    c = jnp.sqrt(jnp.sum(wc * wc))
    wc = jnp.where(c > 1.0, wc / c, wc)
    wv = jnp.stack([wp[0, 0] * wc, wq[0, 0] * wc, wr[0, 0] * wc,
                    jnp.zeros((EMBED,), jnp.float32)])

    tp, sp, tq, sq, tr, sr = _build_tc_sweep(n)(wv, P.T, Q.T, R.T)

    ps2 = ps.astype(jnp.int32).reshape(-1, IDX_CHUNK)
    qs2 = qs.astype(jnp.int32).reshape(-1, IDX_CHUNK)
    rs2 = rs.astype(jnp.int32).reshape(-1, IDX_CHUNK)

    y, ss = _build_sc_lookup(batch)(ps2, qs2, rs2, tp, sp, tq, sq, tr, sr)

    bias = (bp[0] + bq[0] + br[0]) * jnp.sum(wc)
    inferences = (y + bias).reshape(batch, 1)
    regs = REG_COEF * (jnp.sqrt(jnp.sum(ss[:, 0]))
                       + jnp.sqrt(jnp.sum(ss[:, 1]))
                       + jnp.sqrt(jnp.sum(ss[:, 2])))
    return (inferences, regs)


# speedup vs baseline: 2.7660x; 1.1918x over previous
import functools

import jax
import jax.numpy as jnp
from jax import lax
from jax.experimental import pallas as pl
from jax.experimental.pallas import tpu as pltpu
from jax.experimental.pallas import tpu_sc as plsc

EMBED = 16
IDX_CHUNK = 128
LANE_BLK = 32768
REG_COEF = 0.001


def _tc_sweep_body(wv, pt, qt, rt, tp, sp, tq, sq, tr, sr):
    one = jnp.ones((1, EMBED), jnp.float32)
    for t, (src, tdst, sdst) in enumerate(
        ((pt, tp, sp), (qt, tq, sq), (rt, tr, sr))):
        blk = src[...]
        w = wv[t:t + 1, :]
        tdst[...] = jnp.dot(w, blk, preferred_element_type=jnp.float32)[0]
        sdst[...] = jnp.dot(one, blk * blk,
                            preferred_element_type=jnp.float32)[0]


@functools.lru_cache(maxsize=None)
def _build_tc_sweep(n: int):
    grid = (pl.cdiv(n, LANE_BLK),)
    tbl_spec = pl.BlockSpec((EMBED, LANE_BLK), lambda c: (0, c),
                            pipeline_mode=pl.Buffered(buffer_count=2))
    vec_spec = pl.BlockSpec((LANE_BLK,), lambda c: (c,),
                            pipeline_mode=pl.Buffered(buffer_count=2))
    return pl.pallas_call(
        _tc_sweep_body,
        grid=grid,
        in_specs=[pl.BlockSpec((4, EMBED), lambda c: (0, 0)),
                  tbl_spec, tbl_spec, tbl_spec],
        out_specs=[vec_spec] * 6,
        out_shape=[jax.ShapeDtypeStruct((n,), jnp.float32)] * 6,
        compiler_params=pltpu.CompilerParams(
            dimension_semantics=("arbitrary",)),
    )


@functools.lru_cache(maxsize=None)
def _build_sc_lookup(batch: int):
    info = plsc.get_sparse_core_info()
    ncores, nsub, lanes = info.num_cores, info.num_subcores, info.num_lanes
    nw = ncores * nsub
    bpw = batch // nw
    nch = bpw // IDX_CHUNK
    nblk = bpw // lanes

    mesh = plsc.VectorSubcoreMesh(core_axis_name="c", subcore_axis_name="s")

    @functools.partial(
        pl.kernel,
        mesh=mesh,
        out_type=[
            jax.ShapeDtypeStruct((batch,), jnp.float32),
            jax.ShapeDtypeStruct((nw, EMBED), jnp.float32),
        ],
        scratch_types=[
            pltpu.VMEM((nch, IDX_CHUNK), jnp.int32),
            pltpu.VMEM((nch, IDX_CHUNK), jnp.int32),
            pltpu.VMEM((nch, IDX_CHUNK), jnp.int32),
            pltpu.VMEM((bpw,), jnp.float32),
            pltpu.VMEM((bpw,), jnp.float32),
            pltpu.VMEM((bpw,), jnp.float32),
            pltpu.VMEM((bpw,), jnp.float32),
            pltpu.VMEM((bpw,), jnp.float32),
            pltpu.VMEM((bpw,), jnp.float32),
            pltpu.VMEM((bpw,), jnp.float32),
            pltpu.VMEM((EMBED,), jnp.float32),
            pltpu.SemaphoreType.DMA,
        ],
        compiler_params=pltpu.CompilerParams(
            needs_layout_passes=False, use_tc_tiling_on_sc=False),
    )
    def sc_lookup(ps2, qs2, rs2, tp, sp, tq, sq, tr, sr,
                  out_y, out_ss,
                  idx_p, idx_q, idx_r,
                  g_tp, g_sp, g_tq, g_sq, g_tr, g_sr,
                  y_v, ss_v, sem):
        wid = lax.axis_index("s") * ncores + lax.axis_index("c")
        base = wid * bpw

        pltpu.sync_copy(ps2.at[pl.ds(wid * nch, nch)], idx_p)
        pltpu.sync_copy(qs2.at[pl.ds(wid * nch, nch)], idx_q)
        pltpu.sync_copy(rs2.at[pl.ds(wid * nch, nch)], idx_r)

        copies = []
        for vec, idx, dst in ((tp, idx_p, g_tp), (sp, idx_p, g_sp),
                              (tq, idx_q, g_tq), (sq, idx_q, g_sq),
                              (tr, idx_r, g_tr), (sr, idx_r, g_sr)):
            for j in range(nch):
                copies.append(pltpu.async_copy(
                    vec.at[idx.at[j]],
                    dst.at[pl.ds(j * IDX_CHUNK, IDX_CHUNK)],
                    sem))
        for c in copies:
            c.wait()

        iota = lax.iota(jnp.int32, lanes)
        zero = jnp.zeros((lanes,), jnp.float32)

        def body(b, carry):
            ap, aq, ar = carry
            sl = pl.ds(b * lanes, lanes)
            y_v[sl] = g_tp[sl] + g_tq[sl] + g_tr[sl]
            return ap + g_sp[sl], aq + g_sq[sl], ar + g_sr[sl]

        ap, aq, ar = lax.fori_loop(0, nblk, body, (zero, zero, zero))

        ssvec = (jnp.where(iota == 0, jnp.sum(ap), 0.0)
                 + jnp.where(iota == 1, jnp.sum(aq), 0.0)
                 + jnp.where(iota == 2, jnp.sum(ar), 0.0))
        ss_v[...] = ssvec.astype(jnp.float32)

        pltpu.sync_copy(y_v, out_y.at[pl.ds(base, bpw)])
        pltpu.sync_copy(ss_v, out_ss.at[wid])

    return sc_lookup


def kernel(ps, qs, rs, P, Q, R, wp, bp, wq, bq, wr, br, W):
    batch = ps.shape[0]
    n = P.shape[0]
    wc = W[0].astype(jnp.float32)
    c = jnp.sqrt(jnp.sum(wc * wc))
    wc = jnp.where(c > 1.0, wc / c, wc)
    wv = jnp.stack([wp[0, 0] * wc, wq[0, 0] * wc, wr[0, 0] * wc,
                    jnp.zeros((EMBED,), jnp.float32)])

    tp, sp, tq, sq, tr, sr = _build_tc_sweep(n)(wv, P.T, Q.T, R.T)

    ps2 = ps.astype(jnp.int32).reshape(-1, IDX_CHUNK)
    qs2 = qs.astype(jnp.int32).reshape(-1, IDX_CHUNK)
    rs2 = rs.astype(jnp.int32).reshape(-1, IDX_CHUNK)

    y, ss = _build_sc_lookup(batch)(ps2, qs2, rs2, tp, sp, tq, sq, tr, sr)

    bias = (bp[0] + bq[0] + br[0]) * jnp.sum(wc)
    inferences = (y + bias).reshape(batch, 1)
    regs = REG_COEF * (jnp.sqrt(jnp.sum(ss[:, 0]))
                       + jnp.sqrt(jnp.sum(ss[:, 1]))
                       + jnp.sqrt(jnp.sum(ss[:, 2])))
    return (inferences, regs)


# speedup vs baseline: 2.8578x; 1.0332x over previous
import functools

import jax
import jax.numpy as jnp
from jax import lax
from jax.experimental import pallas as pl
from jax.experimental.pallas import tpu as pltpu
from jax.experimental.pallas import tpu_sc as plsc

EMBED = 16
IDX_CHUNK = 128
LANE_BLK = 65536
REG_COEF = 0.001


def _tc_sweep_body(wv, pt, qt, rt, tp, sp, tq, sq, tr, sr):
    one = jnp.ones((1, EMBED), jnp.float32)
    for t, (src, tdst, sdst) in enumerate(
        ((pt, tp, sp), (qt, tq, sq), (rt, tr, sr))):
        blk = src[...]
        w = wv[t:t + 1, :]
        tdst[...] = jnp.dot(w, blk, preferred_element_type=jnp.float32)[0]
        sdst[...] = jnp.dot(one, blk * blk,
                            preferred_element_type=jnp.float32)[0]


@functools.lru_cache(maxsize=None)
def _build_tc_sweep(n: int):
    grid = (pl.cdiv(n, LANE_BLK),)
    tbl_spec = pl.BlockSpec((EMBED, LANE_BLK), lambda c: (0, c),
                            pipeline_mode=pl.Buffered(buffer_count=2))
    vec_spec = pl.BlockSpec((LANE_BLK,), lambda c: (c,),
                            pipeline_mode=pl.Buffered(buffer_count=2))
    return pl.pallas_call(
        _tc_sweep_body,
        grid=grid,
        in_specs=[pl.BlockSpec((4, EMBED), lambda c: (0, 0)),
                  tbl_spec, tbl_spec, tbl_spec],
        out_specs=[vec_spec] * 6,
        out_shape=[jax.ShapeDtypeStruct((n,), jnp.float32)] * 6,
        compiler_params=pltpu.CompilerParams(
            dimension_semantics=("arbitrary",)),
    )


@functools.lru_cache(maxsize=None)
def _build_sc_lookup(batch: int):
    info = plsc.get_sparse_core_info()
    ncores, nsub, lanes = info.num_cores, info.num_subcores, info.num_lanes
    nw = ncores * nsub
    bpw = batch // nw
    nch = bpw // IDX_CHUNK
    nblk = bpw // lanes

    mesh = plsc.VectorSubcoreMesh(core_axis_name="c", subcore_axis_name="s")

    @functools.partial(
        pl.kernel,
        mesh=mesh,
        out_type=[
            jax.ShapeDtypeStruct((batch,), jnp.float32),
            jax.ShapeDtypeStruct((nw, EMBED), jnp.float32),
        ],
        scratch_types=[
            pltpu.VMEM((nch, IDX_CHUNK), jnp.int32),
            pltpu.VMEM((nch, IDX_CHUNK), jnp.int32),
            pltpu.VMEM((nch, IDX_CHUNK), jnp.int32),
            pltpu.VMEM((bpw,), jnp.float32),
            pltpu.VMEM((bpw,), jnp.float32),
            pltpu.VMEM((bpw,), jnp.float32),
            pltpu.VMEM((bpw,), jnp.float32),
            pltpu.VMEM((bpw,), jnp.float32),
            pltpu.VMEM((bpw,), jnp.float32),
            pltpu.VMEM((bpw,), jnp.float32),
            pltpu.VMEM((EMBED,), jnp.float32),
            pltpu.SemaphoreType.DMA,
        ],
        compiler_params=pltpu.CompilerParams(
            needs_layout_passes=False, use_tc_tiling_on_sc=False),
    )
    def sc_lookup(ps2, qs2, rs2, tp, sp, tq, sq, tr, sr,
                  out_y, out_ss,
                  idx_p, idx_q, idx_r,
                  g_tp, g_sp, g_tq, g_sq, g_tr, g_sr,
                  y_v, ss_v, sem):
        wid = lax.axis_index("s") * ncores + lax.axis_index("c")
        base = wid * bpw

        pltpu.sync_copy(ps2.at[pl.ds(wid * nch, nch)], idx_p)
        pltpu.sync_copy(qs2.at[pl.ds(wid * nch, nch)], idx_q)
        pltpu.sync_copy(rs2.at[pl.ds(wid * nch, nch)], idx_r)

        copies = []
        for vec, idx, dst in ((tp, idx_p, g_tp), (sp, idx_p, g_sp),
                              (tq, idx_q, g_tq), (sq, idx_q, g_sq),
                              (tr, idx_r, g_tr), (sr, idx_r, g_sr)):
            for j in range(nch):
                copies.append(pltpu.async_copy(
                    vec.at[idx.at[j]],
                    dst.at[pl.ds(j * IDX_CHUNK, IDX_CHUNK)],
                    sem))
        for c in copies:
            c.wait()

        iota = lax.iota(jnp.int32, lanes)
        zero = jnp.zeros((lanes,), jnp.float32)

        def body(b, carry):
            ap, aq, ar = carry
            sl = pl.ds(b * lanes, lanes)
            y_v[sl] = g_tp[sl] + g_tq[sl] + g_tr[sl]
            return ap + g_sp[sl], aq + g_sq[sl], ar + g_sr[sl]

        ap, aq, ar = lax.fori_loop(0, nblk, body, (zero, zero, zero))

        ssvec = (jnp.where(iota == 0, jnp.sum(ap), 0.0)
                 + jnp.where(iota == 1, jnp.sum(aq), 0.0)
                 + jnp.where(iota == 2, jnp.sum(ar), 0.0))
        ss_v[...] = ssvec.astype(jnp.float32)

        pltpu.sync_copy(y_v, out_y.at[pl.ds(base, bpw)])
        pltpu.sync_copy(ss_v, out_ss.at[wid])

    return sc_lookup


def kernel(ps, qs, rs, P, Q, R, wp, bp, wq, bq, wr, br, W):
    batch = ps.shape[0]
    n = P.shape[0]
    wc = W[0].astype(jnp.float32)
    c = jnp.sqrt(jnp.sum(wc * wc))
    wc = jnp.where(c > 1.0, wc / c, wc)
    wv = jnp.stack([wp[0, 0] * wc, wq[0, 0] * wc, wr[0, 0] * wc,
                    jnp.zeros((EMBED,), jnp.float32)])

    tp, sp, tq, sq, tr, sr = _build_tc_sweep(n)(wv, P.T, Q.T, R.T)

    ps2 = ps.astype(jnp.int32).reshape(-1, IDX_CHUNK)
    qs2 = qs.astype(jnp.int32).reshape(-1, IDX_CHUNK)
    rs2 = rs.astype(jnp.int32).reshape(-1, IDX_CHUNK)

    y, ss = _build_sc_lookup(batch)(ps2, qs2, rs2, tp, sp, tq, sq, tr, sr)

    bias = (bp[0] + bq[0] + br[0]) * jnp.sum(wc)
    inferences = (y + bias).reshape(batch, 1)
    regs = REG_COEF * (jnp.sqrt(jnp.sum(ss[:, 0]))
                       + jnp.sqrt(jnp.sum(ss[:, 1]))
                       + jnp.sqrt(jnp.sum(ss[:, 2])))
    return (inferences, regs)


# speedup vs baseline: 2.9078x; 1.0175x over previous
import functools

import jax
import jax.numpy as jnp
from jax import lax
from jax.experimental import pallas as pl
from jax.experimental.pallas import tpu as pltpu
from jax.experimental.pallas import tpu_sc as plsc

EMBED = 16
IDX_CHUNK = 128
LANE_BLK = 65536
REG_COEF = 0.001


def _pack_bf16_pair(hi_f32, lo_f32):
    hi = lax.bitcast_convert_type(hi_f32, jnp.int32)
    lo = lax.bitcast_convert_type(lo_f32, jnp.int32)
    hi = (hi + 0x8000) & ~0xFFFF
    lo = lax.shift_right_logical(lo + 0x8000, 16)
    return hi | lo


def _tc_sweep_body(wv, pt, qt, rt, op, oq, orr):
    one = jnp.ones((1, EMBED), jnp.float32)
    for t, (src, dst) in enumerate(((pt, op), (qt, oq), (rt, orr))):
        blk = src[...]
        w = wv[t:t + 1, :]
        tv = jnp.dot(w, blk, preferred_element_type=jnp.float32)[0]
        sv = jnp.dot(one, blk * blk, preferred_element_type=jnp.float32)[0]
        dst[...] = _pack_bf16_pair(tv, sv)


@functools.lru_cache(maxsize=None)
def _build_tc_sweep(n: int):
    grid = (pl.cdiv(n, LANE_BLK),)
    tbl_spec = pl.BlockSpec((EMBED, LANE_BLK), lambda c: (0, c),
                            pipeline_mode=pl.Buffered(buffer_count=2))
    vec_spec = pl.BlockSpec((LANE_BLK,), lambda c: (c,),
                            pipeline_mode=pl.Buffered(buffer_count=2))
    return pl.pallas_call(
        _tc_sweep_body,
        grid=grid,
        in_specs=[pl.BlockSpec((4, EMBED), lambda c: (0, 0)),
                  tbl_spec, tbl_spec, tbl_spec],
        out_specs=[vec_spec] * 3,
        out_shape=[jax.ShapeDtypeStruct((n,), jnp.int32)] * 3,
        compiler_params=pltpu.CompilerParams(
            dimension_semantics=("arbitrary",)),
    )


@functools.lru_cache(maxsize=None)
def _build_sc_lookup(batch: int):
    info = plsc.get_sparse_core_info()
    ncores, nsub, lanes = info.num_cores, info.num_subcores, info.num_lanes
    nw = ncores * nsub
    bpw = batch // nw
    nch = bpw // IDX_CHUNK
    nblk = bpw // lanes

    mesh = plsc.VectorSubcoreMesh(core_axis_name="c", subcore_axis_name="s")

    @functools.partial(
        pl.kernel,
        mesh=mesh,
        out_type=[
            jax.ShapeDtypeStruct((batch,), jnp.float32),
            jax.ShapeDtypeStruct((nw, EMBED), jnp.float32),
        ],
        scratch_types=[
            pltpu.VMEM((nch, IDX_CHUNK), jnp.int32),
            pltpu.VMEM((nch, IDX_CHUNK), jnp.int32),
            pltpu.VMEM((nch, IDX_CHUNK), jnp.int32),
            pltpu.VMEM((bpw,), jnp.int32),
            pltpu.VMEM((bpw,), jnp.int32),
            pltpu.VMEM((bpw,), jnp.int32),
            pltpu.VMEM((bpw,), jnp.float32),
            pltpu.VMEM((EMBED,), jnp.float32),
            pltpu.SemaphoreType.DMA,
        ],
        compiler_params=pltpu.CompilerParams(
            needs_layout_passes=False, use_tc_tiling_on_sc=False),
    )
    def sc_lookup(ps2, qs2, rs2, vp, vq, vr,
                  out_y, out_ss,
                  idx_p, idx_q, idx_r, g_p, g_q, g_r,
                  y_v, ss_v, sem):
        wid = lax.axis_index("s") * ncores + lax.axis_index("c")
        base = wid * bpw

        pltpu.sync_copy(ps2.at[pl.ds(wid * nch, nch)], idx_p)
        pltpu.sync_copy(qs2.at[pl.ds(wid * nch, nch)], idx_q)
        pltpu.sync_copy(rs2.at[pl.ds(wid * nch, nch)], idx_r)

        copies = []
        for vec, idx, dst in ((vp, idx_p, g_p), (vq, idx_q, g_q),
                              (vr, idx_r, g_r)):
            for j in range(nch):
                copies.append(pltpu.async_copy(
                    vec.at[idx.at[j]],
                    dst.at[pl.ds(j * IDX_CHUNK, IDX_CHUNK)],
                    sem))
        for c in copies:
            c.wait()

        iota = lax.iota(jnp.int32, lanes)
        zero = jnp.zeros((lanes,), jnp.float32)
        himask = jnp.full((lanes,), ~0xFFFF, jnp.int32)
        sixteen = jnp.full((lanes,), 16, jnp.int32)

        def unpack(word):
            t = lax.bitcast_convert_type(word & himask, jnp.float32)
            s = lax.bitcast_convert_type(
                lax.shift_left(word, sixteen), jnp.float32)
            return t, s

        def body(b, carry):
            ap, aq, ar = carry
            sl = pl.ds(b * lanes, lanes)
            tP, sP = unpack(g_p[sl])
            tQ, sQ = unpack(g_q[sl])
            tR, sR = unpack(g_r[sl])
            y_v[sl] = tP + tQ + tR
            return ap + sP, aq + sQ, ar + sR

        ap, aq, ar = lax.fori_loop(0, nblk, body, (zero, zero, zero))

        ssvec = (jnp.where(iota == 0, jnp.sum(ap), 0.0)
                 + jnp.where(iota == 1, jnp.sum(aq), 0.0)
                 + jnp.where(iota == 2, jnp.sum(ar), 0.0))
        ss_v[...] = ssvec.astype(jnp.float32)

        pltpu.sync_copy(y_v, out_y.at[pl.ds(base, bpw)])
        pltpu.sync_copy(ss_v, out_ss.at[wid])

    return sc_lookup


def kernel(ps, qs, rs, P, Q, R, wp, bp, wq, bq, wr, br, W):
    batch = ps.shape[0]
    n = P.shape[0]
    wc = W[0].astype(jnp.float32)
    c = jnp.sqrt(jnp.sum(wc * wc))
    wc = jnp.where(c > 1.0, wc / c, wc)
    wv = jnp.stack([wp[0, 0] * wc, wq[0, 0] * wc, wr[0, 0] * wc,
                    jnp.zeros((EMBED,), jnp.float32)])

    vp, vq, vr = _build_tc_sweep(n)(wv, P.T, Q.T, R.T)

    ps2 = ps.astype(jnp.int32).reshape(-1, IDX_CHUNK)
    qs2 = qs.astype(jnp.int32).reshape(-1, IDX_CHUNK)
    rs2 = rs.astype(jnp.int32).reshape(-1, IDX_CHUNK)

    y, ss = _build_sc_lookup(batch)(ps2, qs2, rs2, vp, vq, vr)

    bias = (bp[0] + bq[0] + br[0]) * jnp.sum(wc)
    inferences = (y + bias).reshape(batch, 1)
    regs = REG_COEF * (jnp.sqrt(jnp.sum(ss[:, 0]))
                       + jnp.sqrt(jnp.sum(ss[:, 1]))
                       + jnp.sqrt(jnp.sum(ss[:, 2])))
    return (inferences, regs)
